# baseline (speedup 1.0000x reference)
import jax
import jax.numpy as jnp
from jax import lax
from jax.experimental import pallas as pl
from jax.experimental.pallas import tpu as pltpu
from jax.experimental.pallas import tpu_sc as plsc

TOTAL_TOKENS = 32768
D_MODEL = 4096
BATCH = 16
HALF = D_MODEL // 2
LANES = 16
UNROLL = 8


_GATHER_DNUMS = lax.GatherDimensionNumbers(
    offset_dims=(), collapsed_slice_dims=(0,), start_index_map=(0,))


def _permute(x, idx):
    return lax.gather(x, idx[:, None], _GATHER_DNUMS, slice_sizes=(1,),
                      mode=lax.GatherScatterMode.PROMISE_IN_BOUNDS)


def _allreduce_sum(x):
    lane = lax.iota(jnp.int32, 16)
    for d in (1, 2, 4, 8):
        x = x + _permute(x, lane ^ d)
    return x


def _body(hs_hbm, lens_hbm, out_hbm, lens_v, x_v):
    s = lax.axis_index("s")
    b = s

    pltpu.sync_copy(lens_hbm, lens_v)
    lens = lens_v[...].astype(jnp.float32)
    lane = lax.iota(jnp.int32, 16)
    masked = jnp.where(lane <= b, lens, 0.0)
    r_vec = (_allreduce_sum(masked) - 1.0).astype(jnp.int32)
    r = r_vec[0]

    pltpu.sync_copy(hs_hbm.at[r], x_v)

    def ss_step(i, accs):
        base = i * (UNROLL * LANES)
        loaded = [x_v[pl.ds(base + j * LANES, LANES)] for j in range(UNROLL)]
        return tuple(accs[j] + loaded[j] * loaded[j] for j in range(UNROLL))

    zeros = tuple(jnp.zeros((LANES,), jnp.float32) for _ in range(UNROLL))
    accs = lax.fori_loop(0, D_MODEL // (UNROLL * LANES), ss_step, zeros)
    acc = accs[0]
    for a in accs[1:]:
        acc = acc + a
    ssb = _allreduce_sum(acc)

    ssb = jnp.maximum(ssb, 1e-30)
    bits = lax.bitcast_convert_type(ssb, jnp.int32)
    y = lax.bitcast_convert_type(0x5F3759DF - (bits >> 1), jnp.float32)
    for _ in range(3):
        y = y * (1.5 - 0.5 * ssb * y * y)
    norm = ssb * y
    inv = 1.0 / jnp.maximum(norm, 1e-12)

    def sc_step(i, carry):
        base = i * (UNROLL * LANES)
        for j in range(UNROLL):
            ix = pl.ds(base + j * LANES, LANES)
            x_v[ix] = x_v[ix] * inv
        return carry

    lax.fori_loop(0, D_MODEL // (UNROLL * LANES), sc_step, 0)
    pltpu.sync_copy(x_v, out_hbm.at[b])


_pooler = pl.kernel(
    _body,
    out_type=jax.ShapeDtypeStruct((BATCH, D_MODEL), jnp.float32),
    mesh=plsc.VectorSubcoreMesh(core_axis_name="c", subcore_axis_name="s",
                                num_cores=1, num_subcores=16),
    compiler_params=pltpu.CompilerParams(use_tc_tiling_on_sc=True,
                                         skip_device_barrier=True),
    scratch_types=[
        pltpu.VMEM((16,), jnp.int32),
        pltpu.VMEM((D_MODEL,), jnp.float32),
    ],
)


@jax.jit
def kernel(hidden_states, prompt_lens):
    return _pooler(hidden_states, prompt_lens)


# speedup vs baseline: 1.0055x; 1.0055x over previous
import jax
import jax.numpy as jnp
from jax import lax
from jax.experimental import pallas as pl
from jax.experimental.pallas import tpu as pltpu
from jax.experimental.pallas import tpu_sc as plsc

D_MODEL = 4096
BATCH = 16
LANES = 16
UNROLL = 8


_GATHER_DNUMS = lax.GatherDimensionNumbers(
    offset_dims=(), collapsed_slice_dims=(0,), start_index_map=(0,))


def _permute(x, idx):
    return lax.gather(x, idx[:, None], _GATHER_DNUMS, slice_sizes=(1,),
                      mode=lax.GatherScatterMode.PROMISE_IN_BOUNDS)


def _allreduce_sum(x):
    lane = lax.iota(jnp.int32, 16)
    for d in (1, 2, 4, 8):
        x = x + _permute(x, lane ^ d)
    return x


def _body(hs_hbm, lens_hbm, out_hbm, lens_v, x_v):
    b = lax.axis_index("s")

    pltpu.sync_copy(lens_hbm, lens_v)
    lens = lens_v[...].astype(jnp.float32)
    lane = lax.iota(jnp.int32, 16)
    masked = jnp.where(lane <= b, lens, 0.0)
    r_vec = (_allreduce_sum(masked) - 1.0).astype(jnp.int32)
    r = r_vec[0]

    pltpu.sync_copy(hs_hbm.at[r], x_v)

    def ss_step(i, accs):
        base = i * (UNROLL * LANES)
        loaded = [x_v[pl.ds(base + j * LANES, LANES)] for j in range(UNROLL)]
        return tuple(accs[j] + loaded[j] * loaded[j] for j in range(UNROLL))

    zeros = tuple(jnp.zeros((LANES,), jnp.float32) for _ in range(UNROLL))
    accs = lax.fori_loop(0, D_MODEL // (UNROLL * LANES), ss_step, zeros)
    acc = accs[0]
    for a in accs[1:]:
        acc = acc + a
    ssb = _allreduce_sum(acc)

    ssb = jnp.maximum(ssb, 1e-30)
    bits = lax.bitcast_convert_type(ssb, jnp.int32)
    y = lax.bitcast_convert_type(0x5F3759DF - (bits >> 1), jnp.float32)
    for _ in range(3):
        y = y * (1.5 - 0.5 * ssb * y * y)
    norm = ssb * y
    inv = 1.0 / jnp.maximum(norm, 1e-12)

    def sc_step(i, carry):
        base = i * (UNROLL * LANES)
        for j in range(UNROLL):
            ix = pl.ds(base + j * LANES, LANES)
            x_v[ix] = x_v[ix] * inv
        return carry

    lax.fori_loop(0, D_MODEL // (UNROLL * LANES), sc_step, 0)
    pltpu.sync_copy(x_v, out_hbm.at[b])


_pooler = pl.kernel(
    _body,
    out_type=jax.ShapeDtypeStruct((BATCH, D_MODEL), jnp.float32),
    mesh=plsc.VectorSubcoreMesh(core_axis_name="c", subcore_axis_name="s",
                                num_cores=1, num_subcores=16),
    compiler_params=pltpu.CompilerParams(use_tc_tiling_on_sc=True),
    scratch_types=[
        pltpu.VMEM((16,), jnp.int32),
        pltpu.VMEM((D_MODEL,), jnp.float32),
    ],
)


@jax.jit
def kernel(hidden_states, prompt_lens):
    return _pooler(hidden_states, prompt_lens)
